# Initial kernel scaffold; baseline (speedup 1.0000x reference)
import jax
import jax.numpy as jnp
from jax.experimental import pallas as pl


def kernel(in_pc, target_pc):
    raise NotImplementedError("write your pallas kernel here")


# speedup vs baseline: 246.4633x; 246.4633x over previous
import jax
import jax.numpy as jnp
from jax.experimental import pallas as pl
from jax.experimental.pallas import tpu as pltpu

_B = 4
_N = 4096
_ROW_TILE = 512


def _chamfer_body(x_ref, y_ref, out_ref):
    b = pl.program_id(0)

    x = x_ref[0]
    y = y_ref[0]

    xb = x.astype(jnp.bfloat16).astype(jnp.float32)
    yb = y.astype(jnp.bfloat16).astype(jnp.float32)

    y0 = yb[0:1, :]
    y1 = yb[1:2, :]
    y2 = yb[2:3, :]
    ny = y[0:1, :] ** 2 + y[1:2, :] ** 2 + y[2:3, :] ** 2

    n_tiles = _N // _ROW_TILE

    row_sum = jnp.float32(0.0)
    col_min = jnp.full((1, _N), jnp.inf, dtype=jnp.float32)
    for t in range(n_tiles):
        lo = t * _ROW_TILE
        x0 = xb[0:1, lo:lo + _ROW_TILE].reshape(_ROW_TILE, 1)
        x1 = xb[1:2, lo:lo + _ROW_TILE].reshape(_ROW_TILE, 1)
        x2 = xb[2:3, lo:lo + _ROW_TILE].reshape(_ROW_TILE, 1)
        nx = (
            x_ref[0, 0:1, lo:lo + _ROW_TILE] ** 2
            + x_ref[0, 1:2, lo:lo + _ROW_TILE] ** 2
            + x_ref[0, 2:3, lo:lo + _ROW_TILE] ** 2
        ).reshape(_ROW_TILE, 1)
        prod = x0 * y0 + x1 * y1 + x2 * y2
        dist = (-2.0 * prod + nx) + ny
        row_min = jnp.min(dist, axis=1)
        row_sum = row_sum + jnp.sum(row_min)
        col_min = jnp.minimum(col_min, jnp.min(dist, axis=0, keepdims=True))
    batch_total = row_sum + jnp.sum(col_min)

    @pl.when(b == 0)
    def _init():
        out_ref[0, 0] = batch_total

    @pl.when(b != 0)
    def _acc():
        out_ref[0, 0] = out_ref[0, 0] + batch_total


def kernel(in_pc, target_pc):
    total = pl.pallas_call(
        _chamfer_body,
        grid=(_B,),
        in_specs=[
            pl.BlockSpec((1, 3, _N), lambda b: (b, 0, 0)),
            pl.BlockSpec((1, 3, _N), lambda b: (b, 0, 0)),
        ],
        out_specs=pl.BlockSpec(
            (1, 1), lambda b: (0, 0), memory_space=pltpu.SMEM
        ),
        out_shape=jax.ShapeDtypeStruct((1, 1), jnp.float32),
    )(in_pc, target_pc)
    return total[0, 0] / (2.0 * _B * _N)


# speedup vs baseline: 473.9322x; 1.9229x over previous
import jax
import jax.numpy as jnp
from jax.experimental import pallas as pl
from jax.experimental.pallas import tpu as pltpu

_B = 4
_N = 4096
_ROW_TILE = 512


def _chamfer_body(x_ref, y_ref, xt_bf_ref, y_bf_ref, out_ref):
    b = pl.program_id(0)

    y = y_ref[0]
    y_bf = y_bf_ref[0]
    ny = y[0:1, :] ** 2 + y[1:2, :] ** 2 + y[2:3, :] ** 2

    row_sum = jnp.float32(0.0)
    col_min = jnp.full((1, _N), jnp.inf, dtype=jnp.float32)
    for t in range(_N // _ROW_TILE):
        lo = t * _ROW_TILE
        xt = xt_bf_ref[0, lo:lo + _ROW_TILE, :]
        nx = (
            x_ref[0, 0:1, lo:lo + _ROW_TILE] ** 2
            + x_ref[0, 1:2, lo:lo + _ROW_TILE] ** 2
            + x_ref[0, 2:3, lo:lo + _ROW_TILE] ** 2
        ).reshape(_ROW_TILE, 1)
        prod = jax.lax.dot_general(
            xt, y_bf,
            dimension_numbers=(((1,), (0,)), ((), ())),
            preferred_element_type=jnp.float32,
        )
        dist = (-2.0 * prod + nx) + ny
        row_min = jnp.min(dist, axis=1)
        row_sum = row_sum + jnp.sum(row_min)
        col_min = jnp.minimum(col_min, jnp.min(dist, axis=0, keepdims=True))

    batch_total = row_sum + jnp.sum(col_min)

    @pl.when(b == 0)
    def _init():
        out_ref[0, 0] = batch_total

    @pl.when(b != 0)
    def _acc():
        out_ref[0, 0] = out_ref[0, 0] + batch_total


def kernel(in_pc, target_pc):
    in_t_bf = jnp.transpose(in_pc, (0, 2, 1)).astype(jnp.bfloat16)
    tgt_bf = target_pc.astype(jnp.bfloat16)
    total = pl.pallas_call(
        _chamfer_body,
        grid=(_B,),
        in_specs=[
            pl.BlockSpec((1, 3, _N), lambda b: (b, 0, 0)),
            pl.BlockSpec((1, 3, _N), lambda b: (b, 0, 0)),
            pl.BlockSpec((1, _N, 3), lambda b: (b, 0, 0)),
            pl.BlockSpec((1, 3, _N), lambda b: (b, 0, 0)),
        ],
        out_specs=pl.BlockSpec(
            (1, 1), lambda b: (0, 0), memory_space=pltpu.SMEM
        ),
        out_shape=jax.ShapeDtypeStruct((1, 1), jnp.float32),
    )(in_pc, target_pc, in_t_bf, tgt_bf)
    return total[0, 0] / (2.0 * _B * _N)


# speedup vs baseline: 522.2156x; 1.1019x over previous
import jax
import jax.numpy as jnp
from jax.experimental import pallas as pl
from jax.experimental.pallas import tpu as pltpu

_B = 4
_N = 4096
_ROW_TILE = 512


def _chamfer_body(xt_ref, y_ref, xt_bf_ref, y_bf_ref, out_ref):
    b = pl.program_id(0)

    xs = xt_ref[0]
    nx = jnp.sum(xs * xs, axis=1, keepdims=True)
    nxhi = nx.astype(jnp.bfloat16)
    nxlo = (nx - nxhi.astype(jnp.float32)).astype(jnp.bfloat16)
    a_aug = jnp.concatenate(
        [
            xt_bf_ref[0],
            nxhi,
            nxlo,
            jnp.ones((_N, 2), dtype=jnp.bfloat16),
        ],
        axis=1,
    )

    y = y_ref[0]
    ny = y[0:1, :] ** 2 + y[1:2, :] ** 2 + y[2:3, :] ** 2
    nyhi = ny.astype(jnp.bfloat16)
    nylo = (ny - nyhi.astype(jnp.float32)).astype(jnp.bfloat16)
    b_aug = jnp.concatenate(
        [
            jnp.bfloat16(-2.0) * y_bf_ref[0],
            jnp.ones((2, _N), dtype=jnp.bfloat16),
            nyhi,
            nylo,
        ],
        axis=0,
    )

    row_sum = jnp.float32(0.0)
    col_min = jnp.full((1, _N), jnp.inf, dtype=jnp.float32)
    for t in range(_N // _ROW_TILE):
        lo = t * _ROW_TILE
        dist = jax.lax.dot_general(
            a_aug[lo:lo + _ROW_TILE, :], b_aug,
            dimension_numbers=(((1,), (0,)), ((), ())),
            preferred_element_type=jnp.float32,
        )
        row_min = jnp.min(dist, axis=1)
        row_sum = row_sum + jnp.sum(row_min)
        col_min = jnp.minimum(col_min, jnp.min(dist, axis=0, keepdims=True))

    batch_total = row_sum + jnp.sum(col_min)

    @pl.when(b == 0)
    def _init():
        out_ref[0, 0] = batch_total

    @pl.when(b != 0)
    def _acc():
        out_ref[0, 0] = out_ref[0, 0] + batch_total


def kernel(in_pc, target_pc):
    in_t = jnp.transpose(in_pc, (0, 2, 1))
    in_t_bf = in_t.astype(jnp.bfloat16)
    tgt_bf = target_pc.astype(jnp.bfloat16)
    total = pl.pallas_call(
        _chamfer_body,
        grid=(_B,),
        in_specs=[
            pl.BlockSpec((1, _N, 3), lambda b: (b, 0, 0)),
            pl.BlockSpec((1, 3, _N), lambda b: (b, 0, 0)),
            pl.BlockSpec((1, _N, 3), lambda b: (b, 0, 0)),
            pl.BlockSpec((1, 3, _N), lambda b: (b, 0, 0)),
        ],
        out_specs=pl.BlockSpec(
            (1, 1), lambda b: (0, 0), memory_space=pltpu.SMEM
        ),
        out_shape=jax.ShapeDtypeStruct((1, 1), jnp.float32),
    )(in_t, target_pc, in_t_bf, tgt_bf)
    return total[0, 0] / (2.0 * _B * _N)


# speedup vs baseline: 639.0398x; 1.2237x over previous
import jax
import jax.numpy as jnp
from jax.experimental import pallas as pl
from jax.experimental.pallas import tpu as pltpu

_B = 4
_N = 4096
_ROW_TILE = 512


def _chamfer_body(x_ref, y_ref, out_ref):
    total = jnp.float32(0.0)
    for b in range(_B):
        x = x_ref[b]
        y = y_ref[b]

        nx = x[0:1, :] ** 2 + x[1:2, :] ** 2 + x[2:3, :] ** 2
        nxhi = nx.astype(jnp.bfloat16)
        nxlo = (nx - nxhi.astype(jnp.float32)).astype(jnp.bfloat16)
        a_aug = jnp.concatenate(
            [
                x.astype(jnp.bfloat16),
                nxhi,
                nxlo,
                jnp.ones((2, _N), dtype=jnp.bfloat16),
            ],
            axis=0,
        )

        ny = y[0:1, :] ** 2 + y[1:2, :] ** 2 + y[2:3, :] ** 2
        nyhi = ny.astype(jnp.bfloat16)
        nylo = (ny - nyhi.astype(jnp.float32)).astype(jnp.bfloat16)
        b_aug = jnp.concatenate(
            [
                jnp.bfloat16(-2.0) * y.astype(jnp.bfloat16),
                jnp.ones((2, _N), dtype=jnp.bfloat16),
                nyhi,
                nylo,
            ],
            axis=0,
        )

        row_sum = jnp.float32(0.0)
        col_min = jnp.full((1, _N), jnp.inf, dtype=jnp.float32)
        for t in range(_N // _ROW_TILE):
            lo = t * _ROW_TILE
            dist = jax.lax.dot_general(
                a_aug[:, lo:lo + _ROW_TILE], b_aug,
                dimension_numbers=(((0,), (0,)), ((), ())),
                preferred_element_type=jnp.float32,
            )
            row_min = jnp.min(dist, axis=1)
            row_sum = row_sum + jnp.sum(row_min)
            col_min = jnp.minimum(
                col_min, jnp.min(dist, axis=0, keepdims=True)
            )
        total = total + row_sum + jnp.sum(col_min)

    out_ref[0, 0] = total * jnp.float32(1.0 / (2.0 * _B * _N))


def kernel(in_pc, target_pc):
    total = pl.pallas_call(
        _chamfer_body,
        out_specs=pl.BlockSpec(memory_space=pltpu.SMEM),
        out_shape=jax.ShapeDtypeStruct((1, 1), jnp.float32),
    )(in_pc, target_pc)
    return total[0, 0]
